# baseline (speedup 1.0000x reference)
import functools

import jax
import jax.numpy as jnp
from jax import lax
from jax.experimental import pallas as pl
from jax.experimental.pallas import tpu as pltpu
from jax.experimental.pallas import tpu_sc as plsc

NN = 10000
EE = 320000
F_IN = 128
HID = 256
HALF = 128

NC = 2
NS = 16
ROWS_PER_TILE = NN // NS


_BK = 80
_DEG_EPT = EE // (NC * NS)
_DEG_NB = _DEG_EPT // _BK


def _edge_pipeline(tab_ref, gidx_hbm, sidx_hbm, acc,
                   isrcs, idsts, isems, dsems, rowss, gsems, base, nb,
                   indirect=True):
    def load_idx(i, bl):
        off = base + i * _BK
        if indirect:
            pltpu.async_copy(gidx_hbm.at[pl.ds(off, _BK)], isrcs[bl],
                             isems[bl])
        pltpu.async_copy(sidx_hbm.at[pl.ds(off, _BK)], idsts[bl], dsems[bl])

    def gather_src(ph):
        return tab_ref.at[isrcs[ph]] if indirect else tab_ref

    def step(i, ph):
        j = i - 2
        bj = (ph - 2) % 3

        @pl.when(jnp.logical_and(j >= 0, j < nb))
        def _():
            pltpu.make_async_copy(gather_src(bj), rowss[bj],
                                  gsems[bj]).wait()
            off = base + j * _BK
            pltpu.make_async_copy(sidx_hbm.at[pl.ds(off, _BK)],
                                  idsts[bj], dsems[bj]).wait()
            pltpu.sync_copy(rowss[bj], acc.at[idsts[bj]], add=True)

        @pl.when(i + 1 < nb)
        def _():
            load_idx(i + 1, (ph + 1) % 3)

        @pl.when(i < nb)
        def _():
            if indirect:
                off = base + i * _BK
                pltpu.make_async_copy(gidx_hbm.at[pl.ds(off, _BK)],
                                      isrcs[ph], isems[ph]).wait()
            pltpu.async_copy(gather_src(ph), rowss[ph], gsems[ph])

    load_idx(0, 0)

    def body(p, _):
        for ph in range(3):
            step(p * 3 + ph, ph)
        return 0

    lax.fori_loop(0, (nb + 2 + 2) // 3, body, 0)


@functools.cache
def _get_deg_kernel():
    mesh = plsc.VectorSubcoreMesh(core_axis_name="c", subcore_axis_name="s",
                                  num_cores=NC, num_subcores=NS)
    return pl.kernel(
        _deg_body,
        out_type=jax.ShapeDtypeStruct((NC, NS, ROWS_PER_TILE, HALF),
                                      jnp.float32),
        mesh=mesh,
        scratch_types=(
            [pltpu.VMEM((_BK,), jnp.int32)] * 3
            + [pltpu.VMEM((_BK, HALF), jnp.float32)] * 3
            + [pltpu.VMEM_SHARED((NN, HALF), jnp.float32)]
            + [pltpu.SemaphoreType.DMA] * 6
        ),
    )


def _deg_body(dst_hbm, ones_hbm, zeros_hbm, out_hbm,
              id0, id1, id2, r0, r1, r2, acc,
              di0, di1, di2, g0, g1, g2):
    c = lax.axis_index("c")
    s = lax.axis_index("s")

    pltpu.sync_copy(zeros_hbm, acc.at[pl.ds(s * ROWS_PER_TILE, ROWS_PER_TILE)])
    plsc.subcore_barrier()

    base = (c * NS + s) * _DEG_EPT
    _edge_pipeline(ones_hbm, None, dst_hbm, acc,
                   (None, None, None), (id0, id1, id2),
                   (None, None, None), (di0, di1, di2),
                   (r0, r1, r2), (g0, g1, g2), base, _DEG_NB,
                   indirect=False)
    plsc.subcore_barrier()

    pltpu.sync_copy(acc.at[pl.ds(s * ROWS_PER_TILE, ROWS_PER_TILE)],
                    out_hbm.at[c].at[s])


_AGG_EPT = EE // NS
_AGG_NB = _AGG_EPT // _BK


@functools.cache
def _get_agg_kernel():
    mesh = plsc.VectorSubcoreMesh(core_axis_name="c", subcore_axis_name="s",
                                  num_cores=NC, num_subcores=NS)
    return pl.kernel(
        _agg_body,
        out_type=jax.ShapeDtypeStruct((NC, NS, ROWS_PER_TILE, HALF), jnp.float32),
        mesh=mesh,
        scratch_types=(
            [pltpu.VMEM((_BK,), jnp.int32)] * 6
            + [pltpu.VMEM((_BK, HALF), jnp.float32)] * 3
            + [pltpu.VMEM_SHARED((NN, HALF), jnp.float32)]
            + [pltpu.SemaphoreType.DMA] * 9
        ),
    )


def _agg_body(hs_lo_hbm, hs_hi_hbm, src_hbm, dst_hbm, zeros_hbm, out_hbm,
              is0, is1, is2, id0, id1, id2, r0, r1, r2, acc,
              gi0, gi1, gi2, di0, di1, di2, g0, g1, g2):
    isrcs = (is0, is1, is2)
    idsts = (id0, id1, id2)
    isems = (gi0, gi1, gi2)
    dsems = (di0, di1, di2)
    rowss = (r0, r1, r2)
    gsems = (g0, g1, g2)
    c = lax.axis_index("c")
    s = lax.axis_index("s")

    pltpu.sync_copy(zeros_hbm, acc.at[pl.ds(s * ROWS_PER_TILE, ROWS_PER_TILE)])
    plsc.subcore_barrier()

    base = s * _AGG_EPT

    def run_edges(hs_ref):
        _edge_pipeline(hs_ref, src_hbm, dst_hbm, acc,
                       isrcs, idsts, isems, dsems, rowss, gsems,
                       base, _AGG_NB)

    pl.when(c == 0)(lambda: run_edges(hs_lo_hbm))
    pl.when(c == 1)(lambda: run_edges(hs_hi_hbm))
    plsc.subcore_barrier()

    pltpu.sync_copy(acc.at[pl.ds(s * ROWS_PER_TILE, ROWS_PER_TILE)],
                    out_hbm.at[c].at[s])


_BN = 2000


def _a0_body(deg_ref, x_ref, w_ref, lo_ref, hi_ref, dinv_ref):
    d = deg_ref[0, :, 0] + deg_ref[1, :, 0] + 1.0
    dv = lax.rsqrt(d)
    hw = jnp.dot(x_ref[...], w_ref[...], preferred_element_type=jnp.float32)
    hs = hw * dv[:, None]
    lo_ref[...] = hs[:, :HALF]
    hi_ref[...] = hs[:, HALF:]
    dinv_ref[...] = dv[:, None]


def _layer0(degs, x, w0):
    return pl.pallas_call(
        _a0_body,
        grid=(NN // _BN,),
        in_specs=[
            pl.BlockSpec((NC, _BN, HALF), lambda i: (0, i, 0)),
            pl.BlockSpec((_BN, F_IN), lambda i: (i, 0)),
            pl.BlockSpec((F_IN, HID), lambda i: (0, 0)),
        ],
        out_specs=[
            pl.BlockSpec((_BN, HALF), lambda i: (i, 0)),
            pl.BlockSpec((_BN, HALF), lambda i: (i, 0)),
            pl.BlockSpec((_BN, 1), lambda i: (i, 0)),
        ],
        out_shape=[
            jax.ShapeDtypeStruct((NN, HALF), jnp.float32),
            jax.ShapeDtypeStruct((NN, HALF), jnp.float32),
            jax.ShapeDtypeStruct((NN, 1), jnp.float32),
        ],
    )(degs, x, w0)


def _amid_body(agg_ref, lo_ref, hi_ref, dinv_ref, b_ref, w_ref,
               olo_ref, ohi_ref):
    dv = dinv_ref[...]
    lo = jax.nn.relu((agg_ref[0] + lo_ref[...]) * dv + b_ref[0, :HALF][None, :])
    hi = jax.nn.relu((agg_ref[1] + hi_ref[...]) * dv + b_ref[0, HALF:][None, :])
    h = jnp.concatenate([lo, hi], axis=1)
    hs = jnp.dot(h, w_ref[...], preferred_element_type=jnp.float32) * dv
    olo_ref[...] = hs[:, :HALF]
    ohi_ref[...] = hs[:, HALF:]


def _layer_mid(agg, hs_lo, hs_hi, dinv, b_prev, w):
    return pl.pallas_call(
        _amid_body,
        grid=(NN // _BN,),
        in_specs=[
            pl.BlockSpec((NC, _BN, HALF), lambda i: (0, i, 0)),
            pl.BlockSpec((_BN, HALF), lambda i: (i, 0)),
            pl.BlockSpec((_BN, HALF), lambda i: (i, 0)),
            pl.BlockSpec((_BN, 1), lambda i: (i, 0)),
            pl.BlockSpec((1, HID), lambda i: (0, 0)),
            pl.BlockSpec((HID, HID), lambda i: (0, 0)),
        ],
        out_specs=[
            pl.BlockSpec((_BN, HALF), lambda i: (i, 0)),
            pl.BlockSpec((_BN, HALF), lambda i: (i, 0)),
        ],
        out_shape=[
            jax.ShapeDtypeStruct((NN, HALF), jnp.float32),
            jax.ShapeDtypeStruct((NN, HALF), jnp.float32),
        ],
    )(agg, hs_lo, hs_hi, dinv, b_prev, w)


def _a3_body(agg_ref, lo_ref, hi_ref, dinv_ref, b_ref, wl_ref, bl_ref,
             out_ref):
    dv = dinv_ref[...]
    lo = jax.nn.relu((agg_ref[0] + lo_ref[...]) * dv + b_ref[0, :HALF][None, :])
    hi = jax.nn.relu((agg_ref[1] + hi_ref[...]) * dv + b_ref[0, HALF:][None, :])
    h = jnp.concatenate([lo, hi], axis=1)
    out_ref[...] = (
        jnp.dot(h, wl_ref[...], preferred_element_type=jnp.float32)
        + bl_ref[0, 0]
    )


def _layer_final(agg, hs_lo, hs_hi, dinv, b2, wl, bl):
    return pl.pallas_call(
        _a3_body,
        grid=(NN // _BN,),
        in_specs=[
            pl.BlockSpec((NC, _BN, HALF), lambda i: (0, i, 0)),
            pl.BlockSpec((_BN, HALF), lambda i: (i, 0)),
            pl.BlockSpec((_BN, HALF), lambda i: (i, 0)),
            pl.BlockSpec((_BN, 1), lambda i: (i, 0)),
            pl.BlockSpec((1, HID), lambda i: (0, 0)),
            pl.BlockSpec((HID, 1), lambda i: (0, 0)),
            pl.BlockSpec((1, 1), lambda i: (0, 0)),
        ],
        out_specs=pl.BlockSpec((_BN, 1), lambda i: (i, 0)),
        out_shape=jax.ShapeDtypeStruct((NN, 1), jnp.float32),
    )(agg, hs_lo, hs_hi, dinv, b2, wl, bl)


def kernel(x, edge_index, W0, b0, W1, b1, W2, b2, Wl, bl):
    src = edge_index[0].astype(jnp.int32)
    dst = edge_index[1].astype(jnp.int32)

    ones128 = jnp.ones((_BK, HALF), jnp.float32)
    zeros128 = jnp.zeros((ROWS_PER_TILE, HALF), jnp.float32)

    degs = _get_deg_kernel()(dst, ones128, zeros128).reshape(NC, NN, HALF)
    hs_lo, hs_hi, dinv = _layer0(degs, x, W0)

    agg = _get_agg_kernel()(hs_lo, hs_hi, src, dst, zeros128).reshape(
        NC, NN, HALF)
    hs_lo, hs_hi = _layer_mid(agg, hs_lo, hs_hi, dinv, b0.reshape(1, -1), W1)

    agg = _get_agg_kernel()(hs_lo, hs_hi, src, dst, zeros128).reshape(
        NC, NN, HALF)
    hs_lo, hs_hi = _layer_mid(agg, hs_lo, hs_hi, dinv, b1.reshape(1, -1), W2)

    agg = _get_agg_kernel()(hs_lo, hs_hi, src, dst, zeros128).reshape(
        NC, NN, HALF)
    out = _layer_final(agg, hs_lo, hs_hi, dinv, b2.reshape(1, -1), Wl,
                       bl.reshape(1, 1))
    return out.reshape(-1)


# speedup vs baseline: 1.3451x; 1.3451x over previous
import functools

import jax
import jax.numpy as jnp
from jax import lax
from jax.experimental import pallas as pl
from jax.experimental.pallas import tpu as pltpu
from jax.experimental.pallas import tpu_sc as plsc

NN = 10000
EE = 320000
F_IN = 128
HID = 256
HALF = 128

NC = 2
NS = 16
ROWS_PER_TILE = NN // NS


_BK = 100
_NBLK = EE // _BK
_DEG_NB = _NBLK // (NC * NS)
_AGG_NB = _NBLK // NS


def _edge_pipeline(tab_ref, eb_hbm, acc, idxs, isems, rowss, gsems,
                   base, nb, indirect=True):
    def load_idx(i, bl):
        pltpu.async_copy(eb_hbm.at[base + i], idxs[bl], isems[bl])

    def step(i, ph):
        j = i - 2
        bj = (ph - 2) % 3

        @pl.when(jnp.logical_and(j >= 0, j < nb))
        def _():
            if indirect:
                pltpu.make_async_copy(tab_ref.at[idxs[bj].at[0]], rowss[bj],
                                      gsems[bj]).wait()
            pltpu.sync_copy(rowss[bj], acc.at[idxs[bj].at[1]], add=True)

        @pl.when(i + 1 < nb)
        def _():
            load_idx(i + 1, (ph + 1) % 3)

        @pl.when(i < nb)
        def _():
            pltpu.make_async_copy(eb_hbm.at[base + i], idxs[ph],
                                  isems[ph]).wait()
            if indirect:
                pltpu.async_copy(tab_ref.at[idxs[ph].at[0]], rowss[ph],
                                 gsems[ph])

    if not indirect:
        for b in range(3):
            pltpu.async_copy(tab_ref, rowss[b], gsems[b])
        for b in range(3):
            pltpu.make_async_copy(tab_ref, rowss[b], gsems[b]).wait()

    load_idx(0, 0)

    def body(p, _):
        for ph in range(3):
            step(p * 3 + ph, ph)
        return 0

    lax.fori_loop(0, (nb + 2 + 2) // 3, body, 0)


@functools.cache
def _get_deg_kernel():
    mesh = plsc.VectorSubcoreMesh(core_axis_name="c", subcore_axis_name="s",
                                  num_cores=NC, num_subcores=NS)
    return pl.kernel(
        _deg_body,
        out_type=jax.ShapeDtypeStruct((NC, NS, ROWS_PER_TILE, HALF),
                                      jnp.float32),
        mesh=mesh,
        scratch_types=(
            [pltpu.VMEM((2, _BK), jnp.int32)] * 3
            + [pltpu.VMEM((_BK, HALF), jnp.float32)] * 3
            + [pltpu.VMEM_SHARED((NN, HALF), jnp.float32)]
            + [pltpu.SemaphoreType.DMA] * 6
        ),
    )


def _deg_body(eb_hbm, ones_hbm, zeros_hbm, out_hbm,
              id0, id1, id2, r0, r1, r2, acc,
              di0, di1, di2, g0, g1, g2):
    c = lax.axis_index("c")
    s = lax.axis_index("s")

    pltpu.sync_copy(zeros_hbm, acc.at[pl.ds(s * ROWS_PER_TILE, ROWS_PER_TILE)])
    plsc.subcore_barrier()

    base = (c * NS + s) * _DEG_NB
    _edge_pipeline(ones_hbm, eb_hbm, acc,
                   (id0, id1, id2), (di0, di1, di2),
                   (r0, r1, r2), (g0, g1, g2), base, _DEG_NB,
                   indirect=False)
    plsc.subcore_barrier()

    pltpu.sync_copy(acc.at[pl.ds(s * ROWS_PER_TILE, ROWS_PER_TILE)],
                    out_hbm.at[c].at[s])


@functools.cache
def _get_agg_kernel():
    mesh = plsc.VectorSubcoreMesh(core_axis_name="c", subcore_axis_name="s",
                                  num_cores=NC, num_subcores=NS)
    return pl.kernel(
        _agg_body,
        out_type=jax.ShapeDtypeStruct((NC, NS, ROWS_PER_TILE, HALF), jnp.float32),
        mesh=mesh,
        scratch_types=(
            [pltpu.VMEM((2, _BK), jnp.int32)] * 3
            + [pltpu.VMEM((_BK, HALF), jnp.float32)] * 3
            + [pltpu.VMEM_SHARED((NN, HALF), jnp.float32)]
            + [pltpu.SemaphoreType.DMA] * 6
        ),
    )


def _agg_body(hs_lo_hbm, hs_hi_hbm, eb_hbm, zeros_hbm, out_hbm,
              id0, id1, id2, r0, r1, r2, acc,
              di0, di1, di2, g0, g1, g2):
    idxs = (id0, id1, id2)
    isems = (di0, di1, di2)
    rowss = (r0, r1, r2)
    gsems = (g0, g1, g2)
    c = lax.axis_index("c")
    s = lax.axis_index("s")

    pltpu.sync_copy(zeros_hbm, acc.at[pl.ds(s * ROWS_PER_TILE, ROWS_PER_TILE)])
    plsc.subcore_barrier()

    base = s * _AGG_NB

    def run_edges(hs_ref):
        _edge_pipeline(hs_ref, eb_hbm, acc, idxs, isems, rowss, gsems,
                       base, _AGG_NB)

    pl.when(c == 0)(lambda: run_edges(hs_lo_hbm))
    pl.when(c == 1)(lambda: run_edges(hs_hi_hbm))
    plsc.subcore_barrier()

    pltpu.sync_copy(acc.at[pl.ds(s * ROWS_PER_TILE, ROWS_PER_TILE)],
                    out_hbm.at[c].at[s])


_BN = 2000


def _a0_body(deg_ref, x_ref, w_ref, lo_ref, hi_ref, dinv_ref):
    d = deg_ref[0, :, 0] + deg_ref[1, :, 0] + 1.0
    dv = lax.rsqrt(d)
    hw = jnp.dot(x_ref[...], w_ref[...], preferred_element_type=jnp.float32)
    hs = hw * dv[:, None]
    lo_ref[...] = hs[:, :HALF]
    hi_ref[...] = hs[:, HALF:]
    dinv_ref[...] = dv[:, None]


def _layer0(degs, x, w0):
    return pl.pallas_call(
        _a0_body,
        grid=(NN // _BN,),
        in_specs=[
            pl.BlockSpec((NC, _BN, HALF), lambda i: (0, i, 0)),
            pl.BlockSpec((_BN, F_IN), lambda i: (i, 0)),
            pl.BlockSpec((F_IN, HID), lambda i: (0, 0)),
        ],
        out_specs=[
            pl.BlockSpec((_BN, HALF), lambda i: (i, 0)),
            pl.BlockSpec((_BN, HALF), lambda i: (i, 0)),
            pl.BlockSpec((_BN, 1), lambda i: (i, 0)),
        ],
        out_shape=[
            jax.ShapeDtypeStruct((NN, HALF), jnp.float32),
            jax.ShapeDtypeStruct((NN, HALF), jnp.float32),
            jax.ShapeDtypeStruct((NN, 1), jnp.float32),
        ],
    )(degs, x, w0)


def _amid_body(agg_ref, lo_ref, hi_ref, dinv_ref, b_ref, w_ref,
               olo_ref, ohi_ref):
    dv = dinv_ref[...]
    lo = jax.nn.relu((agg_ref[0] + lo_ref[...]) * dv + b_ref[0, :HALF][None, :])
    hi = jax.nn.relu((agg_ref[1] + hi_ref[...]) * dv + b_ref[0, HALF:][None, :])
    h = jnp.concatenate([lo, hi], axis=1)
    hs = jnp.dot(h, w_ref[...], preferred_element_type=jnp.float32) * dv
    olo_ref[...] = hs[:, :HALF]
    ohi_ref[...] = hs[:, HALF:]


def _layer_mid(agg, hs_lo, hs_hi, dinv, b_prev, w):
    return pl.pallas_call(
        _amid_body,
        grid=(NN // _BN,),
        in_specs=[
            pl.BlockSpec((NC, _BN, HALF), lambda i: (0, i, 0)),
            pl.BlockSpec((_BN, HALF), lambda i: (i, 0)),
            pl.BlockSpec((_BN, HALF), lambda i: (i, 0)),
            pl.BlockSpec((_BN, 1), lambda i: (i, 0)),
            pl.BlockSpec((1, HID), lambda i: (0, 0)),
            pl.BlockSpec((HID, HID), lambda i: (0, 0)),
        ],
        out_specs=[
            pl.BlockSpec((_BN, HALF), lambda i: (i, 0)),
            pl.BlockSpec((_BN, HALF), lambda i: (i, 0)),
        ],
        out_shape=[
            jax.ShapeDtypeStruct((NN, HALF), jnp.float32),
            jax.ShapeDtypeStruct((NN, HALF), jnp.float32),
        ],
    )(agg, hs_lo, hs_hi, dinv, b_prev, w)


def _a3_body(agg_ref, lo_ref, hi_ref, dinv_ref, b_ref, wl_ref, bl_ref,
             out_ref):
    dv = dinv_ref[...]
    lo = jax.nn.relu((agg_ref[0] + lo_ref[...]) * dv + b_ref[0, :HALF][None, :])
    hi = jax.nn.relu((agg_ref[1] + hi_ref[...]) * dv + b_ref[0, HALF:][None, :])
    h = jnp.concatenate([lo, hi], axis=1)
    out_ref[...] = (
        jnp.dot(h, wl_ref[...], preferred_element_type=jnp.float32)
        + bl_ref[0, 0]
    )


def _layer_final(agg, hs_lo, hs_hi, dinv, b2, wl, bl):
    return pl.pallas_call(
        _a3_body,
        grid=(NN // _BN,),
        in_specs=[
            pl.BlockSpec((NC, _BN, HALF), lambda i: (0, i, 0)),
            pl.BlockSpec((_BN, HALF), lambda i: (i, 0)),
            pl.BlockSpec((_BN, HALF), lambda i: (i, 0)),
            pl.BlockSpec((_BN, 1), lambda i: (i, 0)),
            pl.BlockSpec((1, HID), lambda i: (0, 0)),
            pl.BlockSpec((HID, 1), lambda i: (0, 0)),
            pl.BlockSpec((1, 1), lambda i: (0, 0)),
        ],
        out_specs=pl.BlockSpec((_BN, 1), lambda i: (i, 0)),
        out_shape=jax.ShapeDtypeStruct((NN, 1), jnp.float32),
    )(agg, hs_lo, hs_hi, dinv, b2, wl, bl)


def kernel(x, edge_index, W0, b0, W1, b1, W2, b2, Wl, bl):
    src = edge_index[0].astype(jnp.int32)
    dst = edge_index[1].astype(jnp.int32)
    eb = jnp.stack([src.reshape(_NBLK, _BK), dst.reshape(_NBLK, _BK)], axis=1)

    ones128 = jnp.ones((_BK, HALF), jnp.float32)
    zeros128 = jnp.zeros((ROWS_PER_TILE, HALF), jnp.float32)

    degs = _get_deg_kernel()(eb, ones128, zeros128).reshape(NC, NN, HALF)
    hs_lo, hs_hi, dinv = _layer0(degs, x, W0)

    agg = _get_agg_kernel()(hs_lo, hs_hi, eb, zeros128).reshape(
        NC, NN, HALF)
    hs_lo, hs_hi = _layer_mid(agg, hs_lo, hs_hi, dinv, b0.reshape(1, -1), W1)

    agg = _get_agg_kernel()(hs_lo, hs_hi, eb, zeros128).reshape(
        NC, NN, HALF)
    hs_lo, hs_hi = _layer_mid(agg, hs_lo, hs_hi, dinv, b1.reshape(1, -1), W2)

    agg = _get_agg_kernel()(hs_lo, hs_hi, eb, zeros128).reshape(
        NC, NN, HALF)
    out = _layer_final(agg, hs_lo, hs_hi, dinv, b2.reshape(1, -1), Wl,
                       bl.reshape(1, 1))
    return out.reshape(-1)
